# baseline (speedup 1.0000x reference)
import functools

import jax
import jax.numpy as jnp
from jax import lax
from jax.experimental import pallas as pl
from jax.experimental.pallas import tpu as pltpu
from jax.experimental.pallas import tpu_sc as plsc

N = 16384
B = 16
E = 32768
D_EMB = 300
D_PAD = 304
RNN = 50
RNN_PAD = 64
TREE = 30
NUM_CLASS = 3
SEG = N // B

NC = 2
NS = 16
NW = NC * NS

_MESH = dict(core_axis_name="c", subcore_axis_name="s",
             num_cores=NC, num_subcores=NS)
_SC_PARAMS = pltpu.CompilerParams(use_tc_tiling_on_sc=False)


def _emb_gather(emb_matrix, embed_ids):
    rows_per_tile = N // NW
    chunk = 128
    n_chunks = rows_per_tile // chunk

    @functools.partial(
        pl.kernel,
        out_type=jax.ShapeDtypeStruct((N, D_PAD), jnp.float32),
        mesh=plsc.VectorSubcoreMesh(**_MESH),
        scratch_types=[
            pltpu.VMEM((chunk,), jnp.int32),
            pltpu.VMEM((chunk, D_PAD), jnp.float32),
            pltpu.SemaphoreType.DMA,
        ],
        compiler_params=_SC_PARAMS,
    )
    def gather_kernel(table_hbm, ids_hbm, x_hbm, idx_v, rows_v, sem):
        wid = lax.axis_index("s") * NC + lax.axis_index("c")

        @pl.loop(0, n_chunks)
        def _(k):
            base = wid * rows_per_tile + k * chunk
            pltpu.sync_copy(ids_hbm.at[pl.ds(base, chunk)], idx_v)
            pltpu.async_copy(table_hbm.at[idx_v], rows_v, sem).wait()
            pltpu.sync_copy(rows_v, x_hbm.at[pl.ds(base, chunk)])

    return gather_kernel(emb_matrix, embed_ids)


def _token_mlp(x, w1p, b1p):
    blk = 1024

    def body(x_ref, w_ref, b_ref, h_ref):
        h_ref[...] = jnp.tanh(
            jnp.dot(x_ref[...], w_ref[...], preferred_element_type=jnp.float32)
            + b_ref[...]
        )

    return pl.pallas_call(
        body,
        grid=(N // blk,),
        in_specs=[
            pl.BlockSpec((blk, D_PAD), lambda i: (i, 0)),
            pl.BlockSpec((D_PAD, RNN_PAD), lambda i: (0, 0)),
            pl.BlockSpec((1, RNN_PAD), lambda i: (0, 0)),
        ],
        out_specs=pl.BlockSpec((blk, RNN_PAD), lambda i: (i, 0)),
        out_shape=jax.ShapeDtypeStruct((N, RNN_PAD), jnp.float32),
    )(x, w1p, b1p)


def _edge_agg(h_pad, src, dst):
    edges_per_tile = E // NW
    chunk = 128
    n_chunks = edges_per_tile // chunk
    rows_per_tile = N // NS
    zrows = 128

    @functools.partial(
        pl.kernel,
        out_type=jax.ShapeDtypeStruct((NC * N, RNN_PAD), jnp.float32),
        mesh=plsc.VectorSubcoreMesh(**_MESH),
        scratch_types=[
            pltpu.VMEM((chunk,), jnp.int32),
            pltpu.VMEM((chunk,), jnp.int32),
            pltpu.VMEM((chunk, RNN_PAD), jnp.float32),
            pltpu.VMEM((zrows, RNN_PAD), jnp.float32),
            pltpu.VMEM_SHARED((N, RNN_PAD), jnp.float32),
            pltpu.SemaphoreType.DMA,
        ],
        compiler_params=_SC_PARAMS,
    )
    def edge_kernel(h_hbm, src_hbm, dst_hbm, out_hbm,
                    src_v, dst_v, msgs_v, zero_v, agg_sh, sem):
        c = lax.axis_index("c")
        s = lax.axis_index("s")
        wid = s * NC + c

        @pl.loop(0, zrows)
        def _(i):
            for j in range(RNN_PAD // 16):
                zero_v[i, pl.ds(j * 16, 16)] = jnp.zeros((16,), jnp.float32)

        @pl.loop(0, rows_per_tile // zrows)
        def _(k):
            pltpu.sync_copy(zero_v, agg_sh.at[pl.ds(s * rows_per_tile + k * zrows, zrows)])

        plsc.subcore_barrier()

        @pl.loop(0, n_chunks)
        def _(k):
            base = wid * edges_per_tile + k * chunk
            pltpu.sync_copy(src_hbm.at[pl.ds(base, chunk)], src_v)
            pltpu.sync_copy(dst_hbm.at[pl.ds(base, chunk)], dst_v)
            pltpu.async_copy(h_hbm.at[src_v], msgs_v, sem).wait()
            pltpu.sync_copy(msgs_v, agg_sh.at[dst_v], add=True)

        plsc.subcore_barrier()

        pltpu.sync_copy(
            agg_sh.at[pl.ds(s * rows_per_tile, rows_per_tile)],
            out_hbm.at[pl.ds(c * N + s * rows_per_tile, rows_per_tile)],
        )

    return edge_kernel(h_pad, src, dst)


def _pool_classify(h_pad, agg2, m, w2p, b2, wc, bc):
    def body(h_ref, a_ref, m_ref, w2_ref, b2_ref, wc_ref, bc_ref,
             o_ref, l_ref):
        hs = h_ref[...] + a_ref[0] + a_ref[1]
        h2 = jnp.tanh(
            jnp.dot(hs, w2_ref[...], preferred_element_type=jnp.float32)
            + b2_ref[...]
        )
        mm = m_ref[...]
        sums = jnp.sum(h2 * mm, axis=0, keepdims=True)
        cnt = jnp.sum(mm)
        out = sums / jnp.maximum(cnt, 1.0)
        logit = jnp.dot(out, wc_ref[...], preferred_element_type=jnp.float32) \
            + bc_ref[...]
        o_ref[...] = out.reshape(1, 1, TREE)
        l_ref[...] = logit.reshape(1, 1, NUM_CLASS)

    return pl.pallas_call(
        body,
        grid=(B,),
        in_specs=[
            pl.BlockSpec((SEG, RNN_PAD), lambda i: (i, 0)),
            pl.BlockSpec((2, SEG, RNN_PAD), lambda i: (0, i, 0)),
            pl.BlockSpec((SEG, 1), lambda i: (i, 0)),
            pl.BlockSpec((RNN_PAD, TREE), lambda i: (0, 0)),
            pl.BlockSpec((1, TREE), lambda i: (0, 0)),
            pl.BlockSpec((TREE, NUM_CLASS), lambda i: (0, 0)),
            pl.BlockSpec((1, NUM_CLASS), lambda i: (0, 0)),
        ],
        out_specs=[
            pl.BlockSpec((1, 1, TREE), lambda i: (i, 0, 0)),
            pl.BlockSpec((1, 1, NUM_CLASS), lambda i: (i, 0, 0)),
        ],
        out_shape=[
            jax.ShapeDtypeStruct((B, 1, TREE), jnp.float32),
            jax.ShapeDtypeStruct((B, 1, NUM_CLASS), jnp.float32),
        ],
    )(h_pad, agg2, m, w2p, b2, wc, bc)


def kernel(embed_ids, edge_index, target_mask, sentence_len,
           emb_matrix, W1, b1, W2, b2, Wc, bc):
    del sentence_len

    embp = jnp.pad(emb_matrix, ((0, 0), (0, D_PAD - D_EMB)))
    w1p = jnp.pad(W1, ((0, D_PAD - D_EMB), (0, RNN_PAD - RNN)))
    b1p = jnp.pad(b1, (0, RNN_PAD - RNN)).reshape(1, RNN_PAD)
    w2p = jnp.pad(W2, ((0, RNN_PAD - RNN), (0, 0)))
    b2r = b2.reshape(1, TREE)
    bcr = bc.reshape(1, NUM_CLASS)
    m = target_mask.astype(jnp.float32).reshape(N, 1)
    src = edge_index[0]
    dst = edge_index[1]

    x = _emb_gather(embp, embed_ids)
    h_pad = _token_mlp(x, w1p, b1p)
    agg2 = _edge_agg(h_pad, src, dst).reshape(2, N, RNN_PAD)
    out3, log3 = _pool_classify(h_pad, agg2, m, w2p, b2r, Wc, bcr)
    return (log3.reshape(B, NUM_CLASS), out3.reshape(B, TREE))


# speedup vs baseline: 6.9299x; 6.9299x over previous
import functools

import jax
import jax.numpy as jnp
from jax import lax
from jax.experimental import pallas as pl
from jax.experimental.pallas import tpu as pltpu
from jax.experimental.pallas import tpu_sc as plsc

N = 16384
B = 16
E = 32768
V = 100000
VP = 100096
D_EMB = 300
RNN = 50
F = 128
TREE = 30
NUM_CLASS = 3
SEG = N // B

NC = 2
NS = 16
NW = NC * NS
HALF = N // NC

_MESH = dict(core_axis_name="c", subcore_axis_name="s",
             num_cores=NC, num_subcores=NS)
_SC_PARAMS = pltpu.CompilerParams(use_tc_tiling_on_sc=True,
                                  needs_layout_passes=False)


def _vocab_mlp(embT, w1p, b1p):
    vb = 4352

    def body(x_ref, w_ref, b_ref, o_ref):
        xb = x_ref[...].astype(jnp.bfloat16)
        wb = w_ref[...].astype(jnp.bfloat16)
        o_ref[...] = jnp.tanh(
            lax.dot_general(xb, wb,
                            (((0,), (0,)), ((), ())),
                            preferred_element_type=jnp.float32)
            + b_ref[...]
        )

    return pl.pallas_call(
        body,
        grid=(VP // vb,),
        in_specs=[
            pl.BlockSpec((D_EMB, vb), lambda i: (0, i)),
            pl.BlockSpec((D_EMB, F), lambda i: (0, 0)),
            pl.BlockSpec((1, F), lambda i: (0, 0)),
        ],
        out_specs=pl.BlockSpec((vb, F), lambda i: (i, 0)),
        out_shape=jax.ShapeDtypeStruct((VP, F), jnp.float32),
    )(embT, w1p, b1p)


def _h_gather(H, embed_ids):
    rows_per_tile = N // NW
    chunk = 128
    n_chunks = rows_per_tile // chunk

    @functools.partial(
        pl.kernel,
        out_type=jax.ShapeDtypeStruct((N, F), jnp.float32),
        mesh=plsc.VectorSubcoreMesh(**_MESH),
        scratch_types=[
            pltpu.VMEM((chunk,), jnp.int32),
            pltpu.VMEM((chunk,), jnp.int32),
            pltpu.VMEM((chunk, F), jnp.float32),
            pltpu.VMEM((chunk, F), jnp.float32),
            pltpu.SemaphoreType.DMA,
            pltpu.SemaphoreType.DMA,
        ],
        compiler_params=_SC_PARAMS,
    )
    def gather_kernel(table_hbm, ids_hbm, h_hbm,
                      idx_0, idx_1, rows_0, rows_1, sem0, sem1):
        wid = lax.axis_index("s") * NC + lax.axis_index("c")
        bufs = [(idx_0, rows_0, sem0), (idx_1, rows_1, sem1)]

        def chunk_base(k):
            return wid * rows_per_tile + k * chunk

        gh = {}
        pltpu.sync_copy(ids_hbm.at[pl.ds(chunk_base(0), chunk)], idx_0)
        gh[0] = pltpu.async_copy(table_hbm.at[idx_0], rows_0, sem0)
        for k in range(n_chunks):
            _, rows, _ = bufs[k % 2]
            nidx, nrows, nsem = bufs[(k + 1) % 2]
            if k + 1 < n_chunks:
                pltpu.sync_copy(ids_hbm.at[pl.ds(chunk_base(k + 1), chunk)], nidx)
                gh[k + 1] = pltpu.async_copy(table_hbm.at[nidx], nrows, nsem)
            gh[k].wait()
            pltpu.sync_copy(rows, h_hbm.at[pl.ds(chunk_base(k), chunk)])

    return gather_kernel(H, embed_ids)


def _edge_agg(h_pad, src, dst):
    edges_per_tile = E // NS
    chunk = 128
    n_chunks = edges_per_tile // chunk
    zrows = 32
    rows_per_tile = HALF // NS
    SPARE = 16

    @functools.partial(
        pl.kernel,
        out_type=jax.ShapeDtypeStruct((N, F), jnp.float32),
        mesh=plsc.VectorSubcoreMesh(**_MESH),
        scratch_types=[
            pltpu.VMEM((edges_per_tile,), jnp.int32),
            pltpu.VMEM((edges_per_tile,), jnp.int32),
            pltpu.VMEM((chunk,), jnp.int32),
            pltpu.VMEM((chunk,), jnp.int32),
            pltpu.VMEM((chunk,), jnp.int32),
            pltpu.VMEM((chunk, F), jnp.float32),
            pltpu.VMEM((chunk, F), jnp.float32),
            pltpu.VMEM((chunk, F), jnp.float32),
            pltpu.VMEM((zrows, F), jnp.float32),
            pltpu.VMEM_SHARED((HALF + SPARE, F), jnp.float32),
            pltpu.SemaphoreType.DMA,
            pltpu.SemaphoreType.DMA,
            pltpu.SemaphoreType.DMA,
            pltpu.SemaphoreType.DMA,
            pltpu.SemaphoreType.DMA,
            pltpu.SemaphoreType.DMA,
            pltpu.SemaphoreType.DMA,
        ],
        compiler_params=_SC_PARAMS,
    )
    def edge_kernel(h_hbm, src_hbm, dst_hbm, out_hbm,
                    src1d, dst1d, dv0, dv1, dv2, msgs_0, msgs_1, msgs_2,
                    zero_v, agg_sh, sidx, sg0, sg1, sg2, ss0, ss1, ss2):
        c = lax.axis_index("c")
        s = lax.axis_index("s")
        lo = c * HALF
        trash = HALF + s

        tbase = s * edges_per_tile
        hidx = [
            pltpu.async_copy(src_hbm.at[pl.ds(tbase, edges_per_tile)], src1d, sidx),
            pltpu.async_copy(dst_hbm.at[pl.ds(tbase, edges_per_tile)], dst1d, sidx),
        ]

        @pl.loop(0, zrows)
        def _(i):
            for j in range(F // 16):
                zero_v[i, pl.ds(j * 16, 16)] = jnp.zeros((16,), jnp.float32)

        @pl.loop(0, rows_per_tile // zrows)
        def _(k):
            pltpu.sync_copy(
                zero_v, agg_sh.at[pl.ds(s * rows_per_tile + k * zrows, zrows)])

        @pl.when(s == 0)
        def _():
            pltpu.sync_copy(zero_v.at[pl.ds(0, SPARE)],
                            agg_sh.at[pl.ds(HALF, SPARE)])

        plsc.subcore_barrier()

        for hh in hidx:
            hh.wait()

        msgs = [msgs_0, msgs_1, msgs_2]
        gsem = [sg0, sg1, sg2]
        dvs = [dv0, dv1, dv2]
        ssem = [ss0, ss1, ss2]

        def start_gather(k):
            return pltpu.async_copy(
                h_hbm.at[src1d.at[pl.ds(k * chunk, chunk)]],
                msgs[k % 3], gsem[k % 3])

        gh = {}
        sh = {}
        gh[0] = start_gather(0)
        if n_chunks > 1:
            gh[1] = start_gather(1)
        for k in range(n_chunks):
            if k + 2 < n_chunks:
                if k >= 1:
                    sh[k - 1].wait()
                gh[k + 2] = start_gather(k + 2)
            dv = dvs[k % 3]
            for j in range(chunk // 16):
                d = dst1d[pl.ds(k * chunk + j * 16, 16)]
                dl = d - lo
                ok = (dl >= 0) & (dl < HALF)
                dv[pl.ds(j * 16, 16)] = jnp.where(ok, dl, trash)
            gh[k].wait()
            sh[k] = pltpu.async_copy(msgs[k % 3], agg_sh.at[dv],
                                     ssem[k % 3], add=True)
        for k in range(max(0, n_chunks - 3), n_chunks):
            if k >= 0 and (k + 3 >= n_chunks):
                sh[k].wait()

        plsc.subcore_barrier()

        pltpu.sync_copy(
            agg_sh.at[pl.ds(s * rows_per_tile, rows_per_tile)],
            out_hbm.at[pl.ds(lo + s * rows_per_tile, rows_per_tile)],
        )

    return edge_kernel(h_pad, src, dst)


def _pool_classify(h_pad, agg, m, w2p, b2, wc, bc):
    SPB = 8
    TOK = SPB * SEG

    def body(h_ref, a_ref, m_ref, w2_ref, b2_ref, wc_ref, bc_ref,
             o_ref, l_ref):
        hs = h_ref[...] + a_ref[...]
        h2 = jnp.tanh(
            jnp.dot(hs, w2_ref[...], preferred_element_type=jnp.float32)
            + b2_ref[...]
        )
        mm = m_ref[...]
        rows = lax.broadcasted_iota(jnp.int32, (SPB, TOK), 0)
        cols = lax.broadcasted_iota(jnp.int32, (SPB, TOK), 1)
        sel = (cols // SEG == rows).astype(jnp.float32)
        sums = jnp.dot(sel, h2 * mm, preferred_element_type=jnp.float32)
        cnt = jnp.dot(sel, mm, preferred_element_type=jnp.float32)
        out = sums / jnp.maximum(cnt, 1.0)
        logit = jnp.dot(out, wc_ref[...], preferred_element_type=jnp.float32) \
            + bc_ref[...]
        o_ref[...] = out.reshape(SPB, 1, TREE)
        l_ref[...] = logit.reshape(SPB, 1, NUM_CLASS)

    return pl.pallas_call(
        body,
        grid=(B // SPB,),
        in_specs=[
            pl.BlockSpec((TOK, F), lambda i: (i, 0)),
            pl.BlockSpec((TOK, F), lambda i: (i, 0)),
            pl.BlockSpec((TOK, 1), lambda i: (i, 0)),
            pl.BlockSpec((F, TREE), lambda i: (0, 0)),
            pl.BlockSpec((1, TREE), lambda i: (0, 0)),
            pl.BlockSpec((TREE, NUM_CLASS), lambda i: (0, 0)),
            pl.BlockSpec((1, NUM_CLASS), lambda i: (0, 0)),
        ],
        out_specs=[
            pl.BlockSpec((SPB, 1, TREE), lambda i: (i, 0, 0)),
            pl.BlockSpec((SPB, 1, NUM_CLASS), lambda i: (i, 0, 0)),
        ],
        out_shape=[
            jax.ShapeDtypeStruct((B, 1, TREE), jnp.float32),
            jax.ShapeDtypeStruct((B, 1, NUM_CLASS), jnp.float32),
        ],
    )(h_pad, agg, m, w2p, b2, wc, bc)


def kernel(embed_ids, edge_index, target_mask, sentence_len,
           emb_matrix, W1, b1, W2, b2, Wc, bc):
    del sentence_len

    embT = emb_matrix.T
    w1p = jnp.pad(W1, ((0, 0), (0, F - RNN)))
    b1p = jnp.pad(b1, (0, F - RNN)).reshape(1, F)
    w2p = jnp.pad(W2, ((0, F - RNN), (0, 0)))
    b2r = b2.reshape(1, TREE)
    bcr = bc.reshape(1, NUM_CLASS)
    m = target_mask.astype(jnp.float32).reshape(N, 1)
    src = edge_index[0]
    dst = edge_index[1]

    H = _vocab_mlp(embT, w1p, b1p)
    h_pad = _h_gather(H, embed_ids)
    agg = _edge_agg(h_pad, src, dst)
    out3, log3 = _pool_classify(h_pad, agg, m, w2p, b2r, Wc, bcr)
    return (log3.reshape(B, NUM_CLASS), out3.reshape(B, TREE))


# speedup vs baseline: 7.0142x; 1.0122x over previous
import functools

import jax
import jax.numpy as jnp
from jax import lax
from jax.experimental import pallas as pl
from jax.experimental.pallas import tpu as pltpu
from jax.experimental.pallas import tpu_sc as plsc

N = 16384
B = 16
E = 32768
V = 100000
VP = 100096
D_EMB = 300
RNN = 50
F = 128
TREE = 30
NUM_CLASS = 3
SEG = N // B

NC = 2
NS = 16
NW = NC * NS
HALF = N // NC

_MESH = dict(core_axis_name="c", subcore_axis_name="s",
             num_cores=NC, num_subcores=NS)
_SC_PARAMS = pltpu.CompilerParams(use_tc_tiling_on_sc=True,
                                  needs_layout_passes=False)


def _vocab_mlp(embT, w1p, b1p):
    vb = 5888

    def body(x_ref, w_ref, b_ref, o_ref):
        xb = x_ref[...].astype(jnp.bfloat16)
        wb = w_ref[...].astype(jnp.bfloat16)
        o_ref[...] = jnp.tanh(
            lax.dot_general(xb, wb,
                            (((0,), (0,)), ((), ())),
                            preferred_element_type=jnp.float32)
            + b_ref[...]
        )

    return pl.pallas_call(
        body,
        grid=(VP // vb,),
        in_specs=[
            pl.BlockSpec((D_EMB, vb), lambda i: (0, i)),
            pl.BlockSpec((D_EMB, F), lambda i: (0, 0)),
            pl.BlockSpec((1, F), lambda i: (0, 0)),
        ],
        out_specs=pl.BlockSpec((vb, F), lambda i: (i, 0)),
        out_shape=jax.ShapeDtypeStruct((VP, F), jnp.float32),
    )(embT, w1p, b1p)


def _h_gather(H, embed_ids):
    rows_per_tile = N // NW
    chunk = 128
    n_chunks = rows_per_tile // chunk

    @functools.partial(
        pl.kernel,
        out_type=jax.ShapeDtypeStruct((N, F), jnp.float32),
        mesh=plsc.VectorSubcoreMesh(**_MESH),
        scratch_types=[
            pltpu.VMEM((chunk,), jnp.int32),
            pltpu.VMEM((chunk,), jnp.int32),
            pltpu.VMEM((chunk, F), jnp.float32),
            pltpu.VMEM((chunk, F), jnp.float32),
            pltpu.SemaphoreType.DMA,
            pltpu.SemaphoreType.DMA,
        ],
        compiler_params=_SC_PARAMS,
    )
    def gather_kernel(table_hbm, ids_hbm, h_hbm,
                      idx_0, idx_1, rows_0, rows_1, sem0, sem1):
        wid = lax.axis_index("s") * NC + lax.axis_index("c")
        bufs = [(idx_0, rows_0, sem0), (idx_1, rows_1, sem1)]

        def chunk_base(k):
            return wid * rows_per_tile + k * chunk

        gh = {}
        pltpu.sync_copy(ids_hbm.at[pl.ds(chunk_base(0), chunk)], idx_0)
        gh[0] = pltpu.async_copy(table_hbm.at[idx_0], rows_0, sem0)
        for k in range(n_chunks):
            _, rows, _ = bufs[k % 2]
            nidx, nrows, nsem = bufs[(k + 1) % 2]
            if k + 1 < n_chunks:
                pltpu.sync_copy(ids_hbm.at[pl.ds(chunk_base(k + 1), chunk)], nidx)
                gh[k + 1] = pltpu.async_copy(table_hbm.at[nidx], nrows, nsem)
            gh[k].wait()
            pltpu.sync_copy(rows, h_hbm.at[pl.ds(chunk_base(k), chunk)])

    return gather_kernel(H, embed_ids)


def _edge_agg(h_pad, src, dst):
    edges_per_tile = E // NS
    chunk = 128
    n_chunks = edges_per_tile // chunk
    zrows = 32
    rows_per_tile = HALF // NS
    SPARE = 16

    @functools.partial(
        pl.kernel,
        out_type=jax.ShapeDtypeStruct((N, F), jnp.float32),
        mesh=plsc.VectorSubcoreMesh(**_MESH),
        scratch_types=[
            pltpu.VMEM((edges_per_tile,), jnp.int32),
            pltpu.VMEM((edges_per_tile,), jnp.int32),
            pltpu.VMEM((chunk,), jnp.int32),
            pltpu.VMEM((chunk,), jnp.int32),
            pltpu.VMEM((chunk,), jnp.int32),
            pltpu.VMEM((chunk, F), jnp.float32),
            pltpu.VMEM((chunk, F), jnp.float32),
            pltpu.VMEM((chunk, F), jnp.float32),
            pltpu.VMEM((zrows, F), jnp.float32),
            pltpu.VMEM_SHARED((HALF + SPARE, F), jnp.float32),
            pltpu.SemaphoreType.DMA,
            pltpu.SemaphoreType.DMA,
            pltpu.SemaphoreType.DMA,
            pltpu.SemaphoreType.DMA,
            pltpu.SemaphoreType.DMA,
            pltpu.SemaphoreType.DMA,
            pltpu.SemaphoreType.DMA,
        ],
        compiler_params=_SC_PARAMS,
    )
    def edge_kernel(h_hbm, src_hbm, dst_hbm, out_hbm,
                    src1d, dst1d, dv0, dv1, dv2, msgs_0, msgs_1, msgs_2,
                    zero_v, agg_sh, sidx, sg0, sg1, sg2, ss0, ss1, ss2):
        c = lax.axis_index("c")
        s = lax.axis_index("s")
        lo = c * HALF
        trash = HALF + s

        tbase = s * edges_per_tile
        hidx = [
            pltpu.async_copy(src_hbm.at[pl.ds(tbase, edges_per_tile)], src1d, sidx),
            pltpu.async_copy(dst_hbm.at[pl.ds(tbase, edges_per_tile)], dst1d, sidx),
        ]

        @pl.loop(0, zrows)
        def _(i):
            for j in range(F // 16):
                zero_v[i, pl.ds(j * 16, 16)] = jnp.zeros((16,), jnp.float32)

        @pl.loop(0, rows_per_tile // zrows)
        def _(k):
            pltpu.sync_copy(
                zero_v, agg_sh.at[pl.ds(s * rows_per_tile + k * zrows, zrows)])

        @pl.when(s == 0)
        def _():
            pltpu.sync_copy(zero_v.at[pl.ds(0, SPARE)],
                            agg_sh.at[pl.ds(HALF, SPARE)])

        plsc.subcore_barrier()

        for hh in hidx:
            hh.wait()

        msgs = [msgs_0, msgs_1, msgs_2]
        gsem = [sg0, sg1, sg2]
        dvs = [dv0, dv1, dv2]
        ssem = [ss0, ss1, ss2]

        def start_gather(k):
            return pltpu.async_copy(
                h_hbm.at[src1d.at[pl.ds(k * chunk, chunk)]],
                msgs[k % 3], gsem[k % 3])

        gh = {}
        sh = {}
        gh[0] = start_gather(0)
        if n_chunks > 1:
            gh[1] = start_gather(1)
        for k in range(n_chunks):
            if k + 2 < n_chunks:
                if k >= 1:
                    sh[k - 1].wait()
                gh[k + 2] = start_gather(k + 2)
            dv = dvs[k % 3]
            for j in range(chunk // 16):
                d = dst1d[pl.ds(k * chunk + j * 16, 16)]
                dl = d - lo
                ok = (dl >= 0) & (dl < HALF)
                dv[pl.ds(j * 16, 16)] = jnp.where(ok, dl, trash)
            gh[k].wait()
            sh[k] = pltpu.async_copy(msgs[k % 3], agg_sh.at[dv],
                                     ssem[k % 3], add=True)
        for k in range(max(0, n_chunks - 3), n_chunks):
            if k >= 0 and (k + 3 >= n_chunks):
                sh[k].wait()

        plsc.subcore_barrier()

        pltpu.sync_copy(
            agg_sh.at[pl.ds(s * rows_per_tile, rows_per_tile)],
            out_hbm.at[pl.ds(lo + s * rows_per_tile, rows_per_tile)],
        )

    return edge_kernel(h_pad, src, dst)


def _pool_classify(h_pad, agg, m, w2p, b2, wc, bc):
    SPB = 8
    TOK = SPB * SEG

    def body(h_ref, a_ref, m_ref, w2_ref, b2_ref, wc_ref, bc_ref,
             o_ref, l_ref):
        hs = h_ref[...] + a_ref[...]
        h2 = jnp.tanh(
            jnp.dot(hs, w2_ref[...], preferred_element_type=jnp.float32)
            + b2_ref[...]
        )
        mm = m_ref[...]
        rows = lax.broadcasted_iota(jnp.int32, (SPB, TOK), 0)
        cols = lax.broadcasted_iota(jnp.int32, (SPB, TOK), 1)
        sel = (cols // SEG == rows).astype(jnp.float32)
        sums = jnp.dot(sel, h2 * mm, preferred_element_type=jnp.float32)
        cnt = jnp.dot(sel, mm, preferred_element_type=jnp.float32)
        out = sums / jnp.maximum(cnt, 1.0)
        logit = jnp.dot(out, wc_ref[...], preferred_element_type=jnp.float32) \
            + bc_ref[...]
        o_ref[...] = out.reshape(SPB, 1, TREE)
        l_ref[...] = logit.reshape(SPB, 1, NUM_CLASS)

    return pl.pallas_call(
        body,
        grid=(B // SPB,),
        in_specs=[
            pl.BlockSpec((TOK, F), lambda i: (i, 0)),
            pl.BlockSpec((TOK, F), lambda i: (i, 0)),
            pl.BlockSpec((TOK, 1), lambda i: (i, 0)),
            pl.BlockSpec((F, TREE), lambda i: (0, 0)),
            pl.BlockSpec((1, TREE), lambda i: (0, 0)),
            pl.BlockSpec((TREE, NUM_CLASS), lambda i: (0, 0)),
            pl.BlockSpec((1, NUM_CLASS), lambda i: (0, 0)),
        ],
        out_specs=[
            pl.BlockSpec((SPB, 1, TREE), lambda i: (i, 0, 0)),
            pl.BlockSpec((SPB, 1, NUM_CLASS), lambda i: (i, 0, 0)),
        ],
        out_shape=[
            jax.ShapeDtypeStruct((B, 1, TREE), jnp.float32),
            jax.ShapeDtypeStruct((B, 1, NUM_CLASS), jnp.float32),
        ],
    )(h_pad, agg, m, w2p, b2, wc, bc)


def kernel(embed_ids, edge_index, target_mask, sentence_len,
           emb_matrix, W1, b1, W2, b2, Wc, bc):
    del sentence_len

    embT = emb_matrix.T
    w1p = jnp.pad(W1, ((0, 0), (0, F - RNN)))
    b1p = jnp.pad(b1, (0, F - RNN)).reshape(1, F)
    w2p = jnp.pad(W2, ((0, F - RNN), (0, 0)))
    b2r = b2.reshape(1, TREE)
    bcr = bc.reshape(1, NUM_CLASS)
    m = target_mask.astype(jnp.float32).reshape(N, 1)
    src = edge_index[0]
    dst = edge_index[1]

    H = _vocab_mlp(embT, w1p, b1p)
    h_pad = _h_gather(H, embed_ids)
    agg = _edge_agg(h_pad, src, dst)
    out3, log3 = _pool_classify(h_pad, agg, m, w2p, b2r, Wc, bcr)
    return (log3.reshape(B, NUM_CLASS), out3.reshape(B, TREE))


# speedup vs baseline: 7.0413x; 1.0039x over previous
import functools

import jax
import jax.numpy as jnp
from jax import lax
from jax.experimental import pallas as pl
from jax.experimental.pallas import tpu as pltpu
from jax.experimental.pallas import tpu_sc as plsc

N = 16384
B = 16
E = 32768
V = 100000
VP = 100096
D_EMB = 300
RNN = 50
F = 128
TREE = 30
NUM_CLASS = 3
SEG = N // B

NC = 2
NS = 16
NW = NC * NS
HALF = N // NC

_MESH = dict(core_axis_name="c", subcore_axis_name="s",
             num_cores=NC, num_subcores=NS)
_SC_PARAMS = pltpu.CompilerParams(use_tc_tiling_on_sc=True,
                                  needs_layout_passes=False)


def _vocab_mlp(embT, w1p, b1p):
    vb = 5888

    def body(x_ref, w_ref, b_ref, o_ref):
        xb = x_ref[...].astype(jnp.bfloat16)
        wb = w_ref[...].astype(jnp.bfloat16)
        o_ref[...] = jnp.tanh(
            lax.dot_general(xb, wb,
                            (((0,), (0,)), ((), ())),
                            preferred_element_type=jnp.float32)
            + b_ref[...]
        )

    return pl.pallas_call(
        body,
        grid=(VP // vb,),
        in_specs=[
            pl.BlockSpec((D_EMB, vb), lambda i: (0, i)),
            pl.BlockSpec((D_EMB, F), lambda i: (0, 0)),
            pl.BlockSpec((1, F), lambda i: (0, 0)),
        ],
        out_specs=pl.BlockSpec((vb, F), lambda i: (i, 0)),
        out_shape=jax.ShapeDtypeStruct((VP, F), jnp.float32),
    )(embT, w1p, b1p)


def _h_gather(H, embed_ids):
    rows_per_tile = N // NW
    chunk = 128
    n_chunks = rows_per_tile // chunk

    @functools.partial(
        pl.kernel,
        out_type=jax.ShapeDtypeStruct((N, F), jnp.float32),
        mesh=plsc.VectorSubcoreMesh(**_MESH),
        scratch_types=[
            pltpu.VMEM((chunk,), jnp.int32),
            pltpu.VMEM((chunk,), jnp.int32),
            pltpu.VMEM((chunk, F), jnp.float32),
            pltpu.VMEM((chunk, F), jnp.float32),
            pltpu.SemaphoreType.DMA,
            pltpu.SemaphoreType.DMA,
        ],
        compiler_params=_SC_PARAMS,
    )
    def gather_kernel(table_hbm, ids_hbm, h_hbm,
                      idx_0, idx_1, rows_0, rows_1, sem0, sem1):
        wid = lax.axis_index("s") * NC + lax.axis_index("c")
        bufs = [(idx_0, rows_0, sem0), (idx_1, rows_1, sem1)]

        def chunk_base(k):
            return wid * rows_per_tile + k * chunk

        gh = {}
        pltpu.sync_copy(ids_hbm.at[pl.ds(chunk_base(0), chunk)], idx_0)
        gh[0] = pltpu.async_copy(table_hbm.at[idx_0], rows_0, sem0)
        for k in range(n_chunks):
            _, rows, _ = bufs[k % 2]
            nidx, nrows, nsem = bufs[(k + 1) % 2]
            if k + 1 < n_chunks:
                pltpu.sync_copy(ids_hbm.at[pl.ds(chunk_base(k + 1), chunk)], nidx)
                gh[k + 1] = pltpu.async_copy(table_hbm.at[nidx], nrows, nsem)
            gh[k].wait()
            pltpu.sync_copy(rows, h_hbm.at[pl.ds(chunk_base(k), chunk)])

    return gather_kernel(H, embed_ids)


def _edge_agg(h_pad, src, dst):
    edges_per_tile = E // NS
    chunk = 128
    n_chunks = edges_per_tile // chunk
    zrows = 32
    rows_per_tile = HALF // NS
    SPARE = 16

    @functools.partial(
        pl.kernel,
        out_type=jax.ShapeDtypeStruct((N, F), jnp.float32),
        mesh=plsc.VectorSubcoreMesh(**_MESH),
        scratch_types=[
            pltpu.VMEM((edges_per_tile,), jnp.int32),
            pltpu.VMEM((edges_per_tile,), jnp.int32),
            pltpu.VMEM((chunk,), jnp.int32),
            pltpu.VMEM((chunk,), jnp.int32),
            pltpu.VMEM((chunk,), jnp.int32),
            pltpu.VMEM((chunk, F), jnp.float32),
            pltpu.VMEM((chunk, F), jnp.float32),
            pltpu.VMEM((chunk, F), jnp.float32),
            pltpu.VMEM((zrows, F), jnp.float32),
            pltpu.VMEM_SHARED((HALF + SPARE, F), jnp.float32),
            pltpu.SemaphoreType.DMA,
            pltpu.SemaphoreType.DMA,
            pltpu.SemaphoreType.DMA,
            pltpu.SemaphoreType.DMA,
            pltpu.SemaphoreType.DMA,
            pltpu.SemaphoreType.DMA,
            pltpu.SemaphoreType.DMA,
        ],
        compiler_params=_SC_PARAMS,
    )
    def edge_kernel(h_hbm, src_hbm, dst_hbm, out_hbm,
                    src1d, dst1d, dv0, dv1, dv2, msgs_0, msgs_1, msgs_2,
                    zero_v, agg_sh, sidx, sg0, sg1, sg2, ss0, ss1, ss2):
        c = lax.axis_index("c")
        s = lax.axis_index("s")
        lo = c * HALF
        trash = HALF + s

        tbase = s * edges_per_tile
        hidx = [
            pltpu.async_copy(src_hbm.at[pl.ds(tbase, edges_per_tile)], src1d, sidx),
            pltpu.async_copy(dst_hbm.at[pl.ds(tbase, edges_per_tile)], dst1d, sidx),
        ]

        @pl.loop(0, zrows)
        def _(i):
            for j in range(F // 16):
                zero_v[i, pl.ds(j * 16, 16)] = jnp.zeros((16,), jnp.float32)

        @pl.loop(0, rows_per_tile // zrows)
        def _(k):
            pltpu.sync_copy(
                zero_v, agg_sh.at[pl.ds(s * rows_per_tile + k * zrows, zrows)])

        @pl.when(s == 0)
        def _():
            pltpu.sync_copy(zero_v.at[pl.ds(0, SPARE)],
                            agg_sh.at[pl.ds(HALF, SPARE)])

        plsc.subcore_barrier()

        for hh in hidx:
            hh.wait()

        msgs = [msgs_0, msgs_1, msgs_2]
        gsem = [sg0, sg1, sg2]
        dvs = [dv0, dv1, dv2]
        ssem = [ss0, ss1, ss2]

        def start_gather(k):
            return pltpu.async_copy(
                h_hbm.at[src1d.at[pl.ds(k * chunk, chunk)]],
                msgs[k % 3], gsem[k % 3])

        gh = {}
        sh = {}
        gh[0] = start_gather(0)
        if n_chunks > 1:
            gh[1] = start_gather(1)
        for k in range(n_chunks):
            if k + 2 < n_chunks:
                if k >= 1:
                    sh[k - 1].wait()
                gh[k + 2] = start_gather(k + 2)
            dv = dvs[k % 3]
            for j in range(chunk // 16):
                d = dst1d[pl.ds(k * chunk + j * 16, 16)]
                dl = d - lo
                ok = (dl >= 0) & (dl < HALF)
                dv[pl.ds(j * 16, 16)] = jnp.where(ok, dl, trash)
            gh[k].wait()
            sh[k] = pltpu.async_copy(msgs[k % 3], agg_sh.at[dv],
                                     ssem[k % 3], add=True)
        for k in range(max(0, n_chunks - 3), n_chunks):
            if k >= 0 and (k + 3 >= n_chunks):
                sh[k].wait()

        plsc.subcore_barrier()

        pltpu.sync_copy(
            agg_sh.at[pl.ds(s * rows_per_tile, rows_per_tile)],
            out_hbm.at[pl.ds(lo + s * rows_per_tile, rows_per_tile)],
        )

    return edge_kernel(h_pad, src, dst)


def _pool_classify(h_pad, agg, m, w2p, b2, wc, bc):
    SPB = 4
    TOK = SPB * SEG

    def body(h_ref, a_ref, m_ref, w2_ref, b2_ref, wc_ref, bc_ref,
             o_ref, l_ref):
        hs = h_ref[...] + a_ref[...]
        h2 = jnp.tanh(
            jnp.dot(hs, w2_ref[...], preferred_element_type=jnp.float32)
            + b2_ref[...]
        )
        mm = m_ref[...]
        rows = lax.broadcasted_iota(jnp.int32, (SPB, TOK), 0)
        cols = lax.broadcasted_iota(jnp.int32, (SPB, TOK), 1)
        sel = (cols // SEG == rows).astype(jnp.float32)
        sums = jnp.dot(sel, h2 * mm, preferred_element_type=jnp.float32)
        cnt = jnp.dot(sel, mm, preferred_element_type=jnp.float32)
        out = sums / jnp.maximum(cnt, 1.0)
        logit = jnp.dot(out, wc_ref[...], preferred_element_type=jnp.float32) \
            + bc_ref[...]
        o_ref[...] = out.reshape(SPB, 1, TREE)
        l_ref[...] = logit.reshape(SPB, 1, NUM_CLASS)

    return pl.pallas_call(
        body,
        grid=(B // SPB,),
        in_specs=[
            pl.BlockSpec((TOK, F), lambda i: (i, 0)),
            pl.BlockSpec((TOK, F), lambda i: (i, 0)),
            pl.BlockSpec((TOK, 1), lambda i: (i, 0)),
            pl.BlockSpec((F, TREE), lambda i: (0, 0)),
            pl.BlockSpec((1, TREE), lambda i: (0, 0)),
            pl.BlockSpec((TREE, NUM_CLASS), lambda i: (0, 0)),
            pl.BlockSpec((1, NUM_CLASS), lambda i: (0, 0)),
        ],
        out_specs=[
            pl.BlockSpec((SPB, 1, TREE), lambda i: (i, 0, 0)),
            pl.BlockSpec((SPB, 1, NUM_CLASS), lambda i: (i, 0, 0)),
        ],
        out_shape=[
            jax.ShapeDtypeStruct((B, 1, TREE), jnp.float32),
            jax.ShapeDtypeStruct((B, 1, NUM_CLASS), jnp.float32),
        ],
    )(h_pad, agg, m, w2p, b2, wc, bc)


def kernel(embed_ids, edge_index, target_mask, sentence_len,
           emb_matrix, W1, b1, W2, b2, Wc, bc):
    del sentence_len

    embT = emb_matrix.T
    w1p = jnp.pad(W1, ((0, 0), (0, F - RNN)))
    b1p = jnp.pad(b1, (0, F - RNN)).reshape(1, F)
    w2p = jnp.pad(W2, ((0, F - RNN), (0, 0)))
    b2r = b2.reshape(1, TREE)
    bcr = bc.reshape(1, NUM_CLASS)
    m = target_mask.astype(jnp.float32).reshape(N, 1)
    src = edge_index[0]
    dst = edge_index[1]

    H = _vocab_mlp(embT, w1p, b1p)
    h_pad = _h_gather(H, embed_ids)
    agg = _edge_agg(h_pad, src, dst)
    out3, log3 = _pool_classify(h_pad, agg, m, w2p, b2r, Wc, bcr)
    return (log3.reshape(B, NUM_CLASS), out3.reshape(B, TREE))
